# baseline (speedup 1.0000x reference)
import functools

import jax
import jax.numpy as jnp
from jax import lax
from jax.experimental import pallas as pl
from jax.experimental.pallas import tpu as pltpu
from jax.experimental.pallas import tpu_sc as plsc

N_NODES = 10000
D = 128
N_EDGES = 320000
N_LABEL = 100000

NC = 2
NS = 16
NW = NC * NS
EBLK = 128

NBLK_E = 80
EPW = NBLK_E * EBLK
E_PAD = EPW * NW

N_PAD = 10112
ZROWS = N_PAD // NS
TRASH = N_NODES

NBLK_L = 26
NBLK_L_PAD = 32
LPW = NBLK_L * EBLK
L_PAD = LPW * NW

_mesh = plsc.VectorSubcoreMesh(core_axis_name="c", subcore_axis_name="s")


def _sc_hist(dst3, ones, zeros):
    @functools.partial(
        pl.kernel,
        mesh=_mesh,
        out_type=jax.ShapeDtypeStruct((NC, N_PAD, D), jnp.float32),
        scratch_types=[
            pltpu.VMEM((NBLK_E, EBLK), jnp.int32),
            pltpu.VMEM((EBLK, D), jnp.float32),
            pltpu.VMEM_SHARED((N_PAD, D), jnp.float32),
            pltpu.SemaphoreType.DMA,
        ],
    )
    def k(dst_hbm, ones_hbm, z_hbm, out_hbm, di, ones_v, hist, sem):
        c = lax.axis_index("c")
        s = lax.axis_index("s")
        wid = c * NS + s
        pltpu.sync_copy(dst_hbm.at[wid], di)
        pltpu.sync_copy(ones_hbm, ones_v)
        pltpu.sync_copy(z_hbm.at[pl.ds(s * ZROWS, ZROWS)],
                        hist.at[pl.ds(s * ZROWS, ZROWS)])
        plsc.subcore_barrier()

        @pl.loop(0, NBLK_E, step=8)
        def _(b):
            for j in range(8):
                pltpu.async_copy(ones_v, hist.at[di.at[b + j]], sem, add=True)
            for j in range(8):
                pltpu.make_async_copy(ones_v, hist.at[di.at[b + j]], sem).wait()

        plsc.subcore_barrier()
        pltpu.sync_copy(hist.at[pl.ds(s * ZROWS, ZROWS)],
                        out_hbm.at[c].at[pl.ds(s * ZROWS, ZROWS)])

    return k(dst3, ones, zeros)


def _sc_segsum(table, src3, dst3, zeros):
    @functools.partial(
        pl.kernel,
        mesh=_mesh,
        out_type=jax.ShapeDtypeStruct((NC, N_PAD, D), jnp.float32),
        scratch_types=[
            pltpu.VMEM((NBLK_E // 2, EBLK), jnp.int32),
            pltpu.VMEM((NBLK_E // 2, EBLK), jnp.int32),
            pltpu.VMEM((EBLK, D), jnp.float32),
            pltpu.VMEM((EBLK, D), jnp.float32),
            pltpu.VMEM_SHARED((N_PAD, D), jnp.float32),
            pltpu.SemaphoreType.DMA,
            pltpu.SemaphoreType.DMA,
            pltpu.SemaphoreType.DMA,
            pltpu.SemaphoreType.DMA,
        ],
    )
    def k(tab_hbm, src_hbm, dst_hbm, z_hbm, out_hbm,
          si, di, rows0, rows1, acc, g0, g1, s0, s1):
        c = lax.axis_index("c")
        s = lax.axis_index("s")
        wid = c * NS + s
        HNB = NBLK_E // 2
        pltpu.sync_copy(z_hbm.at[pl.ds(s * ZROWS, ZROWS)],
                        acc.at[pl.ds(s * ZROWS, ZROWS)])
        plsc.subcore_barrier()

        for h in range(2):
            pltpu.sync_copy(src_hbm.at[wid].at[pl.ds(h * HNB, HNB)], si)
            pltpu.sync_copy(dst_hbm.at[wid].at[pl.ds(h * HNB, HNB)], di)
            pltpu.async_copy(tab_hbm.at[si.at[0]], rows0, g0)

            @pl.loop(0, HNB, step=2)
            def _(b):
                pltpu.make_async_copy(tab_hbm.at[si.at[b]], rows0, g0).wait()

                @pl.when(b > 0)
                def _():
                    pltpu.make_async_copy(rows1, acc.at[di.at[b]], s1).wait()

                pltpu.async_copy(tab_hbm.at[si.at[b + 1]], rows1, g1)
                pltpu.async_copy(rows0, acc.at[di.at[b]], s0, add=True)
                pltpu.make_async_copy(tab_hbm.at[si.at[b + 1]], rows1, g1).wait()
                pltpu.make_async_copy(rows0, acc.at[di.at[b]], s0).wait()

                @pl.when(b + 2 < HNB)
                def _():
                    pltpu.async_copy(tab_hbm.at[si.at[b + 2]], rows0, g0)

                pltpu.async_copy(rows1, acc.at[di.at[b + 1]], s1, add=True)

            pltpu.make_async_copy(rows1, acc.at[di.at[0]], s1).wait()

        plsc.subcore_barrier()
        pltpu.sync_copy(acc.at[pl.ds(s * ZROWS, ZROWS)],
                        out_hbm.at[c].at[pl.ds(s * ZROWS, ZROWS)])

    return k(table, src3, dst3, zeros)


def _sc_decode_gather(z, ls3, ld3):
    out_t = jax.ShapeDtypeStruct((L_PAD, D), jnp.float32)

    @functools.partial(
        pl.kernel,
        mesh=_mesh,
        out_type=(out_t, out_t),
        scratch_types=[
            pltpu.VMEM((NBLK_L_PAD, EBLK), jnp.int32),
            pltpu.VMEM((NBLK_L_PAD, EBLK), jnp.int32),
            pltpu.VMEM((EBLK, D), jnp.float32),
            pltpu.VMEM((EBLK, D), jnp.float32),
            pltpu.VMEM((EBLK, D), jnp.float32),
            pltpu.VMEM((EBLK, D), jnp.float32),
            pltpu.SemaphoreType.DMA,
            pltpu.SemaphoreType.DMA,
            pltpu.SemaphoreType.DMA,
            pltpu.SemaphoreType.DMA,
        ],
    )
    def k(z_hbm, s_hbm, d_hbm, os_hbm, od_hbm,
          si, di, sr0, dr0, sr1, dr1, gs0, gd0, gs1, gd1):
        c = lax.axis_index("c")
        s = lax.axis_index("s")
        wid = c * NS + s
        pltpu.sync_copy(s_hbm.at[wid], si)
        pltpu.sync_copy(d_hbm.at[wid], di)
        base = wid * LPW

        pltpu.async_copy(z_hbm.at[si.at[0]], sr0, gs0)
        pltpu.async_copy(z_hbm.at[di.at[0]], dr0, gd0)

        @pl.loop(0, NBLK_L, step=2)
        def _(b):
            off = base + b * EBLK
            pltpu.make_async_copy(z_hbm.at[si.at[b]], sr0, gs0).wait()
            pltpu.make_async_copy(z_hbm.at[di.at[b]], dr0, gd0).wait()
            pltpu.async_copy(z_hbm.at[si.at[b + 1]], sr1, gs1)
            pltpu.async_copy(z_hbm.at[di.at[b + 1]], dr1, gd1)
            pltpu.sync_copy(sr0, os_hbm.at[pl.ds(off, EBLK)])
            pltpu.sync_copy(dr0, od_hbm.at[pl.ds(off, EBLK)])
            pltpu.make_async_copy(z_hbm.at[si.at[b + 1]], sr1, gs1).wait()
            pltpu.make_async_copy(z_hbm.at[di.at[b + 1]], dr1, gd1).wait()

            @pl.when(b + 2 < NBLK_L)
            def _():
                pltpu.async_copy(z_hbm.at[si.at[b + 2]], sr0, gs0)
                pltpu.async_copy(z_hbm.at[di.at[b + 2]], dr0, gd0)

            pltpu.sync_copy(sr1, os_hbm.at[pl.ds(off + EBLK, EBLK)])
            pltpu.sync_copy(dr1, od_hbm.at[pl.ds(off + EBLK, EBLK)])

    return k(z, ls3, ld3)


def _tc_prep(hist, x, W1):
    def body(h_ref, x_ref, w_ref, dinv_ref, y_ref):
        deg = h_ref[0, :, 0] + h_ref[1, :, 0] + 1.0
        dinv = lax.rsqrt(deg)
        dinv_ref[...] = dinv[:, None]
        y_ref[...] = jnp.dot(x_ref[...], w_ref[...],
                             preferred_element_type=jnp.float32) * dinv[:, None]

    return pl.pallas_call(
        body,
        out_shape=(jax.ShapeDtypeStruct((N_PAD, 1), jnp.float32),
                   jax.ShapeDtypeStruct((N_PAD, D), jnp.float32)),
    )(hist, x, W1)


def _tc_mid(acc1, y1, dinv, b1, W2):
    def body(a_ref, y_ref, d_ref, b_ref, w_ref, o_ref):
        dinv = d_ref[...]
        h = jnp.maximum(dinv * (a_ref[0] + a_ref[1] + y_ref[...]) + b_ref[...],
                        0.0)
        o_ref[...] = jnp.dot(h, w_ref[...],
                             preferred_element_type=jnp.float32) * dinv

    return pl.pallas_call(
        body, out_shape=jax.ShapeDtypeStruct((N_PAD, D), jnp.float32),
    )(acc1, y1, dinv, b1, W2)


def _tc_fin(acc2, y2, dinv, b2):
    def body(a_ref, y_ref, d_ref, b_ref, o_ref):
        o_ref[...] = d_ref[...] * (a_ref[0] + a_ref[1] + y_ref[...]) + b_ref[...]

    return pl.pallas_call(
        body, out_shape=jax.ShapeDtypeStruct((N_PAD, D), jnp.float32),
    )(acc2, y2, dinv, b2)


def _tc_dot(zs, zd):
    RB = L_PAD // 8

    def body(a_ref, b_ref, o_ref):
        o_ref[...] = jnp.sum(a_ref[...] * b_ref[...], axis=1).reshape(8, RB // 8)

    return pl.pallas_call(
        body,
        grid=(8,),
        in_specs=[pl.BlockSpec((RB, D), lambda i: (i, 0)),
                  pl.BlockSpec((RB, D), lambda i: (i, 0))],
        out_specs=pl.BlockSpec((8, RB // 8), lambda i: (i, 0)),
        out_shape=jax.ShapeDtypeStruct((64, RB // 8), jnp.float32),
    )(zs, zd)


def _pad_idx_3d(idx, n, nblk, nblk_pad, pad_base, pad_mod):
    per_real = n // NW
    npad = nblk * EBLK - per_real
    real = idx.reshape(NW, per_real)
    padv = pad_base + (jnp.arange(npad, dtype=jnp.int32) % pad_mod)
    pads = jnp.broadcast_to(padv[None, :], (NW, npad))
    p = jnp.concatenate([real, pads], axis=1).reshape(NW, nblk, EBLK)
    if nblk_pad > nblk:
        p = jnp.concatenate(
            [p, jnp.zeros((NW, nblk_pad - nblk, EBLK), jnp.int32)], axis=1)
    return p


def kernel(x, edge_index, edge_label_index, W1, b1, W2, b2):
    src = edge_index[0].astype(jnp.int32)
    dst = edge_index[1].astype(jnp.int32)
    lsrc = edge_label_index[0].astype(jnp.int32)
    ldst = edge_label_index[1].astype(jnp.int32)

    src3 = _pad_idx_3d(src, N_EDGES, NBLK_E, NBLK_E, 0, N_NODES)
    dst3 = _pad_idx_3d(dst, N_EDGES, NBLK_E, NBLK_E, TRASH, N_PAD - N_NODES)
    ls3 = _pad_idx_3d(lsrc, N_LABEL, NBLK_L, NBLK_L_PAD, 0, N_NODES)
    ld3 = _pad_idx_3d(ldst, N_LABEL, NBLK_L, NBLK_L_PAD, 0, N_NODES)

    xp = jnp.concatenate([x, jnp.zeros((N_PAD - N_NODES, D), jnp.float32)])
    ones = jnp.ones((EBLK, D), jnp.float32)
    zeros128 = jnp.zeros((N_PAD, D), jnp.float32)

    hist = _sc_hist(dst3, ones, zeros128)
    dinv, y1 = _tc_prep(hist, xp, W1)
    acc1 = _sc_segsum(y1, src3, dst3, zeros128)
    y2 = _tc_mid(acc1, y1, dinv, b1, W2)
    acc2 = _sc_segsum(y2, src3, dst3, zeros128)
    z = _tc_fin(acc2, y2, dinv, b2)
    zs, zd = _sc_decode_gather(z, ls3, ld3)
    dots = _tc_dot(zs, zd).reshape(NW, LPW)
    return dots[:, :N_LABEL // NW].reshape(N_LABEL)


# speedup vs baseline: 1.1431x; 1.1431x over previous
import dataclasses
import functools

import jax
import jax.numpy as jnp
from jax import lax
from jax.experimental import pallas as pl
from jax.experimental.pallas import tpu as pltpu
from jax.experimental.pallas import tpu_sc as plsc

N_NODES = 10000
D = 128
N_EDGES = 320000
N_LABEL = 100000

NC = 2
NS = 16
NW = NC * NS
EBLK = 128

NBLK_E = 80
EPW = NBLK_E * EBLK
E_PAD = EPW * NW

N_PAD = 10112
ZROWS = N_PAD // NS
TRASH = N_NODES

NBLK_L = 26
NBLK_L_PAD = 32
LPW = NBLK_L * EBLK
L_PAD = LPW * NW

_mesh = plsc.VectorSubcoreMesh(core_axis_name="c", subcore_axis_name="s")

_cp = pltpu.CompilerParams()
if "needs_layout_passes" in pltpu.CompilerParams.__dataclass_fields__:
    _cp = dataclasses.replace(_cp, needs_layout_passes=False)


def _sc_hist(dst3):
    @functools.partial(
        pl.kernel,
        mesh=_mesh,
        out_type=jax.ShapeDtypeStruct((NW, N_PAD), jnp.float32),
        scratch_types=[
            pltpu.VMEM((NBLK_E, EBLK), jnp.int32),
            pltpu.VMEM((N_PAD,), jnp.float32),
        ],
        compiler_params=_cp,
    )
    def k(dst_hbm, out_hbm, di, hist):
        c = lax.axis_index("c")
        s = lax.axis_index("s")
        wid = c * NS + s
        pltpu.sync_copy(dst_hbm.at[wid], di)

        @pl.loop(0, N_PAD, step=16)
        def _(r):
            hist[pl.ds(r, 16)] = jnp.zeros((16,), jnp.float32)

        ones = jnp.ones((16,), jnp.float32)

        @pl.loop(0, NBLK_E)
        def _(b):
            for j in range(8):
                v = di[b, pl.ds(j * 16, 16)]
                plsc.addupdate_scatter(hist, [v], ones)

        pltpu.sync_copy(hist, out_hbm.at[wid])

    return k(dst3)


def _sc_segsum(table, src3, dst3, zeros):
    @functools.partial(
        pl.kernel,
        mesh=_mesh,
        out_type=jax.ShapeDtypeStruct((NC, N_PAD, D), jnp.float32),
        scratch_types=[
            pltpu.VMEM((NBLK_E // 2, EBLK), jnp.int32),
            pltpu.VMEM((NBLK_E // 2, EBLK), jnp.int32),
            pltpu.VMEM((EBLK, D), jnp.float32),
            pltpu.VMEM((EBLK, D), jnp.float32),
            pltpu.VMEM_SHARED((N_PAD, D), jnp.float32),
            pltpu.SemaphoreType.DMA,
            pltpu.SemaphoreType.DMA,
            pltpu.SemaphoreType.DMA,
            pltpu.SemaphoreType.DMA,
        ],
    )
    def k(tab_hbm, src_hbm, dst_hbm, z_hbm, out_hbm,
          si, di, rows0, rows1, acc, g0, g1, s0, s1):
        c = lax.axis_index("c")
        s = lax.axis_index("s")
        wid = c * NS + s
        HNB = NBLK_E // 2
        pltpu.sync_copy(z_hbm.at[pl.ds(s * ZROWS, ZROWS)],
                        acc.at[pl.ds(s * ZROWS, ZROWS)])
        plsc.subcore_barrier()

        for h in range(2):
            pltpu.sync_copy(src_hbm.at[wid].at[pl.ds(h * HNB, HNB)], si)
            pltpu.sync_copy(dst_hbm.at[wid].at[pl.ds(h * HNB, HNB)], di)
            pltpu.async_copy(tab_hbm.at[si.at[0]], rows0, g0)

            @pl.loop(0, HNB, step=2)
            def _(b):
                pltpu.make_async_copy(tab_hbm.at[si.at[b]], rows0, g0).wait()

                @pl.when(b > 0)
                def _():
                    pltpu.make_async_copy(rows1, acc.at[di.at[b]], s1).wait()

                pltpu.async_copy(tab_hbm.at[si.at[b + 1]], rows1, g1)
                pltpu.async_copy(rows0, acc.at[di.at[b]], s0, add=True)
                pltpu.make_async_copy(tab_hbm.at[si.at[b + 1]], rows1, g1).wait()
                pltpu.make_async_copy(rows0, acc.at[di.at[b]], s0).wait()

                @pl.when(b + 2 < HNB)
                def _():
                    pltpu.async_copy(tab_hbm.at[si.at[b + 2]], rows0, g0)

                pltpu.async_copy(rows1, acc.at[di.at[b + 1]], s1, add=True)

            pltpu.make_async_copy(rows1, acc.at[di.at[0]], s1).wait()

        plsc.subcore_barrier()
        pltpu.sync_copy(acc.at[pl.ds(s * ZROWS, ZROWS)],
                        out_hbm.at[c].at[pl.ds(s * ZROWS, ZROWS)])

    return k(table, src3, dst3, zeros)


def _sc_decode_gather(z, ls3, ld3):
    out_t = jax.ShapeDtypeStruct((L_PAD, D), jnp.float32)

    @functools.partial(
        pl.kernel,
        mesh=_mesh,
        out_type=(out_t, out_t),
        scratch_types=[
            pltpu.VMEM((NBLK_L_PAD, EBLK), jnp.int32),
            pltpu.VMEM((NBLK_L_PAD, EBLK), jnp.int32),
            pltpu.VMEM((EBLK, D), jnp.float32),
            pltpu.VMEM((EBLK, D), jnp.float32),
            pltpu.VMEM((EBLK, D), jnp.float32),
            pltpu.VMEM((EBLK, D), jnp.float32),
            pltpu.SemaphoreType.DMA,
            pltpu.SemaphoreType.DMA,
            pltpu.SemaphoreType.DMA,
            pltpu.SemaphoreType.DMA,
        ],
    )
    def k(z_hbm, s_hbm, d_hbm, os_hbm, od_hbm,
          si, di, sr0, dr0, sr1, dr1, gs0, gd0, gs1, gd1):
        c = lax.axis_index("c")
        s = lax.axis_index("s")
        wid = c * NS + s
        pltpu.sync_copy(s_hbm.at[wid], si)
        pltpu.sync_copy(d_hbm.at[wid], di)
        base = wid * LPW

        pltpu.async_copy(z_hbm.at[si.at[0]], sr0, gs0)
        pltpu.async_copy(z_hbm.at[di.at[0]], dr0, gd0)

        @pl.loop(0, NBLK_L, step=2)
        def _(b):
            off = base + b * EBLK
            pltpu.make_async_copy(z_hbm.at[si.at[b]], sr0, gs0).wait()
            pltpu.make_async_copy(z_hbm.at[di.at[b]], dr0, gd0).wait()
            pltpu.async_copy(z_hbm.at[si.at[b + 1]], sr1, gs1)
            pltpu.async_copy(z_hbm.at[di.at[b + 1]], dr1, gd1)
            pltpu.sync_copy(sr0, os_hbm.at[pl.ds(off, EBLK)])
            pltpu.sync_copy(dr0, od_hbm.at[pl.ds(off, EBLK)])
            pltpu.make_async_copy(z_hbm.at[si.at[b + 1]], sr1, gs1).wait()
            pltpu.make_async_copy(z_hbm.at[di.at[b + 1]], dr1, gd1).wait()

            @pl.when(b + 2 < NBLK_L)
            def _():
                pltpu.async_copy(z_hbm.at[si.at[b + 2]], sr0, gs0)
                pltpu.async_copy(z_hbm.at[di.at[b + 2]], dr0, gd0)

            pltpu.sync_copy(sr1, os_hbm.at[pl.ds(off + EBLK, EBLK)])
            pltpu.sync_copy(dr1, od_hbm.at[pl.ds(off + EBLK, EBLK)])

    return k(z, ls3, ld3)


def _tc_prep(hist, x, W1):
    def body(h_ref, x_ref, w_ref, dinv_ref, y_ref):
        deg = jnp.sum(h_ref[...], axis=0) + 1.0
        dinv = lax.rsqrt(deg)
        dinv_ref[...] = dinv[:, None]
        y_ref[...] = jnp.dot(x_ref[...], w_ref[...],
                             preferred_element_type=jnp.float32) * dinv[:, None]

    return pl.pallas_call(
        body,
        out_shape=(jax.ShapeDtypeStruct((N_PAD, 1), jnp.float32),
                   jax.ShapeDtypeStruct((N_PAD, D), jnp.float32)),
    )(hist, x, W1)


def _tc_mid(acc1, y1, dinv, b1, W2):
    def body(a_ref, y_ref, d_ref, b_ref, w_ref, o_ref):
        dinv = d_ref[...]
        h = jnp.maximum(dinv * (a_ref[0] + a_ref[1] + y_ref[...]) + b_ref[...],
                        0.0)
        o_ref[...] = jnp.dot(h, w_ref[...],
                             preferred_element_type=jnp.float32) * dinv

    return pl.pallas_call(
        body, out_shape=jax.ShapeDtypeStruct((N_PAD, D), jnp.float32),
    )(acc1, y1, dinv, b1, W2)


def _tc_fin(acc2, y2, dinv, b2):
    def body(a_ref, y_ref, d_ref, b_ref, o_ref):
        o_ref[...] = d_ref[...] * (a_ref[0] + a_ref[1] + y_ref[...]) + b_ref[...]

    return pl.pallas_call(
        body, out_shape=jax.ShapeDtypeStruct((N_PAD, D), jnp.float32),
    )(acc2, y2, dinv, b2)


def _tc_dot(zs, zd):
    RB = L_PAD // 8

    def body(a_ref, b_ref, o_ref):
        o_ref[...] = jnp.sum(a_ref[...] * b_ref[...], axis=1).reshape(8, RB // 8)

    return pl.pallas_call(
        body,
        grid=(8,),
        in_specs=[pl.BlockSpec((RB, D), lambda i: (i, 0)),
                  pl.BlockSpec((RB, D), lambda i: (i, 0))],
        out_specs=pl.BlockSpec((8, RB // 8), lambda i: (i, 0)),
        out_shape=jax.ShapeDtypeStruct((64, RB // 8), jnp.float32),
    )(zs, zd)


def _pad_idx_3d(idx, n, nblk, nblk_pad, pad_base, pad_mod):
    per_real = n // NW
    npad = nblk * EBLK - per_real
    real = idx.reshape(NW, per_real)
    padv = pad_base + (jnp.arange(npad, dtype=jnp.int32) % pad_mod)
    pads = jnp.broadcast_to(padv[None, :], (NW, npad))
    p = jnp.concatenate([real, pads], axis=1).reshape(NW, nblk, EBLK)
    if nblk_pad > nblk:
        p = jnp.concatenate(
            [p, jnp.zeros((NW, nblk_pad - nblk, EBLK), jnp.int32)], axis=1)
    return p


def kernel(x, edge_index, edge_label_index, W1, b1, W2, b2):
    src = edge_index[0].astype(jnp.int32)
    dst = edge_index[1].astype(jnp.int32)
    lsrc = edge_label_index[0].astype(jnp.int32)
    ldst = edge_label_index[1].astype(jnp.int32)

    src3 = _pad_idx_3d(src, N_EDGES, NBLK_E, NBLK_E, 0, N_NODES)
    dst3 = _pad_idx_3d(dst, N_EDGES, NBLK_E, NBLK_E, TRASH, N_PAD - N_NODES)
    ls3 = _pad_idx_3d(lsrc, N_LABEL, NBLK_L, NBLK_L_PAD, 0, N_NODES)
    ld3 = _pad_idx_3d(ldst, N_LABEL, NBLK_L, NBLK_L_PAD, 0, N_NODES)

    xp = jnp.concatenate([x, jnp.zeros((N_PAD - N_NODES, D), jnp.float32)])
    zeros128 = jnp.zeros((N_PAD, D), jnp.float32)

    hist = _sc_hist(dst3)
    dinv, y1 = _tc_prep(hist, xp, W1)
    acc1 = _sc_segsum(y1, src3, dst3, zeros128)
    y2 = _tc_mid(acc1, y1, dinv, b1, W2)
    acc2 = _sc_segsum(y2, src3, dst3, zeros128)
    z = _tc_fin(acc2, y2, dinv, b2)
    zs, zd = _sc_decode_gather(z, ls3, ld3)
    dots = _tc_dot(zs, zd).reshape(NW, LPW)
    return dots[:, :N_LABEL // NW].reshape(N_LABEL)
